# baseline (speedup 1.0000x reference)
import functools

import jax
from jax._src.config import enable_x64 as _enable_x64
import jax.numpy as jnp
from jax import lax
from jax.experimental import pallas as pl
from jax.experimental.pallas import tpu as pltpu
from jax.experimental.pallas import tpu_sc as plsc

def _sc_compiler_params():
    kw = {"use_tc_tiling_on_sc": False}
    if "needs_layout_passes" in pltpu.CompilerParams.__dataclass_fields__:
        kw["needs_layout_passes"] = False
    return pltpu.CompilerParams(**kw)


D = 64
GPR = 2
NC, NS = 2, 16
NW = NC * NS
CHUNK = 128


def _dequant_lookup(idx2d, weight, scales):
    n_rows = idx2d.shape[0] * idx2d.shape[1]
    rows_per_w = n_rows // NW
    chunks_per_w = rows_per_w // CHUNK

    mesh = plsc.VectorSubcoreMesh(core_axis_name="c", subcore_axis_name="s")

    @functools.partial(
        pl.kernel,
        out_type=jax.ShapeDtypeStruct((n_rows, D), jnp.float32),
        mesh=mesh,
        compiler_params=_sc_compiler_params(),
        scratch_types=[
            pltpu.VMEM((chunks_per_w, CHUNK), jnp.int32),
            pltpu.VMEM((CHUNK, D), jnp.int8),
            pltpu.VMEM((CHUNK, GPR), jnp.float32),
            pltpu.VMEM((CHUNK, D), jnp.float32),
            pltpu.SemaphoreType.DMA,
        ],
    )
    def k(idx_hbm, w_hbm, s_hbm, out_hbm, idx_v, w_v, s_v, o_v, sem):
        i32 = jnp.int32
        wid = lax.axis_index("s") * i32(NC) + lax.axis_index("c")
        base_chunk = wid * i32(chunks_per_w)

        pltpu.sync_copy(idx_hbm.at[pl.ds(base_chunk, chunks_per_w)], idx_v)

        iota = lax.iota(jnp.int32, 16)
        sel = (iota >= i32(8)).astype(jnp.int32)
        col0 = iota * i32(4)

        @pl.loop(0, chunks_per_w)
        def _chunk(j):
            idx_row = idx_v.at[j]
            pltpu.async_copy(w_hbm.at[idx_row], w_v, sem).wait()
            pltpu.async_copy(s_hbm.at[idx_row], s_v, sem).wait()

            @pl.loop(0, CHUNK)
            def _row(r):
                rsplat = jnp.full((16,), r, jnp.int32)
                row32 = plsc.bitcast(w_v[r, :], jnp.int32)
                svec = plsc.load_gather(s_v, [rsplat, sel])
                for kk in range(4):
                    t = (row32 << i32(24 - 8 * kk)) >> i32(24)
                    plsc.store_scatter(
                        o_v, [rsplat, col0 + kk], t.astype(jnp.float32) * svec
                    )

            pltpu.sync_copy(
                o_v, out_hbm.at[pl.ds((base_chunk + j) * i32(CHUNK), CHUNK)]
            )

    return k(idx2d, weight, scales)


def kernel(indices, weight, scales):
    b, l = indices.shape
    idx2d = indices.astype(jnp.int32).reshape(-1, CHUNK)
    with _enable_x64(False):
        out = _dequant_lookup(idx2d, weight, scales)
    return out.reshape(b, l, D)


# speedup vs baseline: 1.0424x; 1.0424x over previous
import functools

import jax
import jax.numpy as jnp
from jax import lax
from jax._src.config import enable_x64 as _enable_x64
from jax.experimental import pallas as pl
from jax.experimental.pallas import tpu as pltpu
from jax.experimental.pallas import tpu_sc as plsc


def _sc_compiler_params():
    kw = {"use_tc_tiling_on_sc": False}
    if "needs_layout_passes" in pltpu.CompilerParams.__dataclass_fields__:
        kw["needs_layout_passes"] = False
    return pltpu.CompilerParams(**kw)


D = 64
W = 16
CW = 18
NC, NS = 2, 16
NW = NC * NS
CHUNK = 128


def _dequant_lookup(idx2d, ctab):
    n_rows = idx2d.shape[0] * idx2d.shape[1]
    rows_per_w = n_rows // NW
    chunks_per_w = rows_per_w // CHUNK

    mesh = plsc.VectorSubcoreMesh(core_axis_name="c", subcore_axis_name="s")

    @functools.partial(
        pl.kernel,
        out_type=jax.ShapeDtypeStruct((n_rows, D), jnp.float32),
        mesh=mesh,
        compiler_params=_sc_compiler_params(),
        scratch_types=[
            pltpu.VMEM((chunks_per_w, CHUNK), jnp.int32),
            pltpu.VMEM((CHUNK, CW), jnp.int32),
            pltpu.VMEM((CHUNK, D), jnp.float32),
            pltpu.SemaphoreType.DMA,
        ],
    )
    def k(idx_hbm, ct_hbm, out_hbm, idx_v, ct_v, o_v, sem):
        i32 = jnp.int32
        wid = lax.axis_index("s") * i32(NC) + lax.axis_index("c")
        base_chunk = wid * i32(chunks_per_w)

        pltpu.sync_copy(idx_hbm.at[pl.ds(base_chunk, chunks_per_w)], idx_v)

        iota = lax.iota(jnp.int32, 16)
        sel = i32(W) + (iota >= i32(8)).astype(jnp.int32)
        col0 = iota * i32(4)

        @pl.loop(0, chunks_per_w)
        def _chunk(j):
            idx_row = idx_v.at[j]
            pltpu.async_copy(ct_hbm.at[idx_row], ct_v, sem).wait()

            @pl.loop(0, CHUNK)
            def _row(r):
                rsplat = jnp.full((16,), r, jnp.int32)
                row32 = ct_v[r, pl.ds(0, 16)]
                svec = plsc.bitcast(
                    plsc.load_gather(ct_v, [rsplat, sel]), jnp.float32
                )
                for kk in range(4):
                    t = (row32 << i32(24 - 8 * kk)) >> i32(24)
                    plsc.store_scatter(
                        o_v, [rsplat, col0 + i32(kk)],
                        t.astype(jnp.float32) * svec,
                    )

            pltpu.sync_copy(
                o_v, out_hbm.at[pl.ds((base_chunk + j) * i32(CHUNK), CHUNK)]
            )

    return k(idx2d, ctab)


def kernel(indices, weight, scales):
    b, l = indices.shape
    v = weight.shape[0]
    idx2d = indices.astype(jnp.int32).reshape(-1, CHUNK)
    with _enable_x64(False):
        w32 = lax.bitcast_convert_type(weight.reshape(v, W, 4), jnp.int32)
        s32 = lax.bitcast_convert_type(scales, jnp.int32)
        ctab = jnp.concatenate([w32, s32], axis=1)
        out = _dequant_lookup(idx2d, ctab)
    return out.reshape(b, l, D)
